# baseline (speedup 1.0000x reference)
import jax
import jax.numpy as jnp
from jax import lax
from jax.experimental import pallas as pl
from jax.experimental.pallas import tpu as pltpu

_B, _C, _S, _N, _D = 8, 16, 4, 256, 16
_DLP, _DM, _NCLS = 16, 16, 2
_CPB = 16


def _vgl_body(adj0_ref, adj1_ref, adj2_ref, adj3_ref, feat_ref, wlp_ref,
              wm1_ref, wm2_ref, wdec_ref, bdec_ref, out_ref, z_scr):
    b = pl.program_id(0)
    cb = pl.program_id(1)
    adj_parts = (adj0_ref, adj1_ref, adj2_ref, adj3_ref)
    npart = _N // len(adj_parts)

    for cc in range(_CPB):
        for s in range(_S):
            fwT = lax.dot_general(wlp_ref[cc, s], feat_ref[0, cc, s],
                                  (((0,), (1,)), ((), ())),
                                  preferred_element_type=jnp.float32)
            for p, apart in enumerate(adj_parts):
                hT = jnp.maximum(
                    lax.dot_general(fwT, apart[0, cc, s],
                                    (((1,), (1,)), ((), ())),
                                    preferred_element_type=jnp.float32),
                    0.0)
                z_scr[pl.ds(cb * _CPB + cc, 1), pl.ds(s * _DLP, _DLP),
                      pl.ds(p * npart, npart)] = hT[None]

    @pl.when(cb == (_C // _CPB) - 1)
    def _end_of_batch():
        z = z_scr[...].reshape(_C, _S * _DLP * _N)
        g = lax.dot_general(z, z, (((1,), (1,)), ((), ())),
                            preferred_element_type=jnp.float32)
        rows = lax.broadcasted_iota(jnp.int32, (_C, _C), 0)
        cols = lax.broadcasted_iota(jnp.int32, (_C, _C), 1)
        eye = (rows == cols).astype(jnp.float32)
        dcol = jnp.sum(g * eye, axis=1, keepdims=True)
        drow = jnp.sum(g * eye, axis=0, keepdims=True)
        denom = (jnp.sqrt(dcol) + 1e-8) * (jnp.sqrt(drow) + 1e-8)
        bg = g / denom
        h1 = jnp.maximum(
            jnp.dot(bg, wm1_ref[...], preferred_element_type=jnp.float32),
            0.0)
        h2 = jnp.maximum(
            jnp.dot(bg, jnp.dot(h1, wm2_ref[...],
                                preferred_element_type=jnp.float32),
                    preferred_element_type=jnp.float32),
            0.0)
        dec = jnp.dot(h2, wdec_ref[...],
                      preferred_element_type=jnp.float32) + bdec_ref[...]
        pooled = jnp.mean(dec, axis=0, keepdims=True)
        out_ref[pl.ds(b, 1), :] = jax.nn.sigmoid(pooled)


def kernel(feats, adjs, W_lp, W_m1, W_m2, W_dec, b_dec):
    b_dec2 = b_dec.reshape(1, _NCLS)
    grid = (_B, _C // _CPB)
    return pl.pallas_call(
        _vgl_body,
        grid=grid,
        in_specs=[
            pl.BlockSpec((1, _CPB, _S, _N // 4, _N),
                         lambda b, c: (b, c, 0, 0, 0)),
            pl.BlockSpec((1, _CPB, _S, _N // 4, _N),
                         lambda b, c: (b, c, 0, 1, 0)),
            pl.BlockSpec((1, _CPB, _S, _N // 4, _N),
                         lambda b, c: (b, c, 0, 2, 0)),
            pl.BlockSpec((1, _CPB, _S, _N // 4, _N),
                         lambda b, c: (b, c, 0, 3, 0)),
            pl.BlockSpec((1, _CPB, _S, _N, _D), lambda b, c: (b, c, 0, 0, 0)),
            pl.BlockSpec((_CPB, _S, _D, _DLP), lambda b, c: (c, 0, 0, 0)),
            pl.BlockSpec((_C, _DM), lambda b, c: (0, 0)),
            pl.BlockSpec((_DM, _DM), lambda b, c: (0, 0)),
            pl.BlockSpec((_DM, _NCLS), lambda b, c: (0, 0)),
            pl.BlockSpec((1, _NCLS), lambda b, c: (0, 0)),
        ],
        out_specs=pl.BlockSpec((_B, _NCLS), lambda b, c: (0, 0)),
        out_shape=jax.ShapeDtypeStruct((_B, _NCLS), jnp.float32),
        scratch_shapes=[
            pltpu.VMEM((_C, _S * _DLP, _N), jnp.float32),
        ],
    )(adjs, adjs, adjs, adjs, feats, W_lp, W_m1, W_m2, W_dec, b_dec2)


# speedup vs baseline: 1.2582x; 1.2582x over previous
import jax
import jax.numpy as jnp
from jax import lax
from jax.experimental import pallas as pl
from jax.experimental.pallas import tpu as pltpu

_B, _C, _S, _N, _D = 8, 16, 4, 256, 16
_DLP, _DM, _NCLS = 16, 16, 2
_CPB = 16


def _vgl_body(adj0_ref, adj1_ref, adj2_ref, adj3_ref, feat_ref, wlp_ref,
              wm1_ref, wm2_ref, wdec_ref, bdec_ref, out_ref, z_scr):
    b = pl.program_id(0)
    cb = pl.program_id(1)
    adj_parts = (adj0_ref, adj1_ref, adj2_ref, adj3_ref)
    cpart = _CPB // len(adj_parts)

    for cc in range(_CPB):
        for s in range(_S):
            fwT = lax.dot_general(wlp_ref[cc, s], feat_ref[0, cc, s],
                                  (((0,), (1,)), ((), ())),
                                  preferred_element_type=jnp.float32)
            adj = adj_parts[cc // cpart][0, cc % cpart, s]
            hT = jnp.maximum(
                lax.dot_general(fwT, adj,
                                (((1,), (1,)), ((), ())),
                                preferred_element_type=jnp.float32),
                0.0)
            z_scr[pl.ds(cb * _CPB + cc, 1), pl.ds(s * _DLP, _DLP), :] = hT[None]

    @pl.when(cb == (_C // _CPB) - 1)
    def _end_of_batch():
        z = z_scr[...].reshape(_C, _S * _DLP * _N)
        g = lax.dot_general(z, z, (((1,), (1,)), ((), ())),
                            preferred_element_type=jnp.float32)
        rows = lax.broadcasted_iota(jnp.int32, (_C, _C), 0)
        cols = lax.broadcasted_iota(jnp.int32, (_C, _C), 1)
        eye = (rows == cols).astype(jnp.float32)
        dcol = jnp.sum(g * eye, axis=1, keepdims=True)
        drow = jnp.sum(g * eye, axis=0, keepdims=True)
        denom = (jnp.sqrt(dcol) + 1e-8) * (jnp.sqrt(drow) + 1e-8)
        bg = g / denom
        h1 = jnp.maximum(
            jnp.dot(bg, wm1_ref[...], preferred_element_type=jnp.float32),
            0.0)
        h2 = jnp.maximum(
            jnp.dot(bg, jnp.dot(h1, wm2_ref[...],
                                preferred_element_type=jnp.float32),
                    preferred_element_type=jnp.float32),
            0.0)
        dec = jnp.dot(h2, wdec_ref[...],
                      preferred_element_type=jnp.float32) + bdec_ref[...]
        pooled = jnp.mean(dec, axis=0, keepdims=True)
        out_ref[pl.ds(b, 1), :] = jax.nn.sigmoid(pooled)


def kernel(feats, adjs, W_lp, W_m1, W_m2, W_dec, b_dec):
    b_dec2 = b_dec.reshape(1, _NCLS)
    grid = (_B, _C // _CPB)
    return pl.pallas_call(
        _vgl_body,
        grid=grid,
        in_specs=[
            pl.BlockSpec((1, _CPB // 4, _S, _N, _N),
                         lambda b, c: (b, 4 * c, 0, 0, 0)),
            pl.BlockSpec((1, _CPB // 4, _S, _N, _N),
                         lambda b, c: (b, 4 * c + 1, 0, 0, 0)),
            pl.BlockSpec((1, _CPB // 4, _S, _N, _N),
                         lambda b, c: (b, 4 * c + 2, 0, 0, 0)),
            pl.BlockSpec((1, _CPB // 4, _S, _N, _N),
                         lambda b, c: (b, 4 * c + 3, 0, 0, 0)),
            pl.BlockSpec((1, _CPB, _S, _N, _D), lambda b, c: (b, c, 0, 0, 0)),
            pl.BlockSpec((_CPB, _S, _D, _DLP), lambda b, c: (c, 0, 0, 0)),
            pl.BlockSpec((_C, _DM), lambda b, c: (0, 0)),
            pl.BlockSpec((_DM, _DM), lambda b, c: (0, 0)),
            pl.BlockSpec((_DM, _NCLS), lambda b, c: (0, 0)),
            pl.BlockSpec((1, _NCLS), lambda b, c: (0, 0)),
        ],
        out_specs=pl.BlockSpec((_B, _NCLS), lambda b, c: (0, 0)),
        out_shape=jax.ShapeDtypeStruct((_B, _NCLS), jnp.float32),
        scratch_shapes=[
            pltpu.VMEM((_C, _S * _DLP, _N), jnp.float32),
        ],
    )(adjs, adjs, adjs, adjs, feats, W_lp, W_m1, W_m2, W_dec, b_dec2)


# speedup vs baseline: 1.2759x; 1.0140x over previous
import jax
import jax.numpy as jnp
from jax import lax
from jax.experimental import pallas as pl
from jax.experimental.pallas import tpu as pltpu

_B, _C, _S, _N, _D = 8, 16, 4, 256, 16
_DLP, _DM, _NCLS = 16, 16, 2


def _vgl_body(adj_ref, feat_ref, wlp_ref, wm1_ref, wm2_ref, wdec_ref,
              bdec_ref, out_ref, z_scr):
    for cc in range(_C):
        for s in range(_S):
            fwT = lax.dot_general(wlp_ref[cc, s], feat_ref[0, cc, s],
                                  (((0,), (1,)), ((), ())),
                                  preferred_element_type=jnp.float32)
            hT = jnp.maximum(
                lax.dot_general(fwT, adj_ref[0, cc, s],
                                (((1,), (1,)), ((), ())),
                                preferred_element_type=jnp.float32),
                0.0)
            z_scr[pl.ds(cc, 1), pl.ds(s * _DLP, _DLP), :] = hT[None]

    z = z_scr[...].reshape(_C, _S * _DLP * _N)
    g = lax.dot_general(z, z, (((1,), (1,)), ((), ())),
                        preferred_element_type=jnp.float32)
    rows = lax.broadcasted_iota(jnp.int32, (_C, _C), 0)
    cols = lax.broadcasted_iota(jnp.int32, (_C, _C), 1)
    eye = (rows == cols).astype(jnp.float32)
    dcol = jnp.sum(g * eye, axis=1, keepdims=True)
    drow = jnp.sum(g * eye, axis=0, keepdims=True)
    denom = (jnp.sqrt(dcol) + 1e-8) * (jnp.sqrt(drow) + 1e-8)
    bg = g / denom
    h1 = jnp.maximum(
        jnp.dot(bg, wm1_ref[...], preferred_element_type=jnp.float32),
        0.0)
    h2 = jnp.maximum(
        jnp.dot(bg, jnp.dot(h1, wm2_ref[...],
                            preferred_element_type=jnp.float32),
                preferred_element_type=jnp.float32),
        0.0)
    dec = jnp.dot(h2, wdec_ref[...],
                  preferred_element_type=jnp.float32) + bdec_ref[...]
    pooled = jnp.mean(dec, axis=0, keepdims=True)
    out_ref[0, pl.ds(0, 1), pl.ds(0, _NCLS)] = jax.nn.sigmoid(pooled)


def kernel(feats, adjs, W_lp, W_m1, W_m2, W_dec, b_dec):
    b_dec2 = b_dec.reshape(1, _NCLS)
    out = pl.pallas_call(
        _vgl_body,
        grid=(_B,),
        in_specs=[
            pl.BlockSpec((1, _C, _S, _N, _N), lambda b: (b, 0, 0, 0, 0)),
            pl.BlockSpec((1, _C, _S, _N, _D), lambda b: (b, 0, 0, 0, 0)),
            pl.BlockSpec((_C, _S, _D, _DLP), lambda b: (0, 0, 0, 0)),
            pl.BlockSpec((_C, _DM), lambda b: (0, 0)),
            pl.BlockSpec((_DM, _DM), lambda b: (0, 0)),
            pl.BlockSpec((_DM, _NCLS), lambda b: (0, 0)),
            pl.BlockSpec((1, _NCLS), lambda b: (0, 0)),
        ],
        out_specs=pl.BlockSpec((1, 8, 128), lambda b: (b, 0, 0)),
        out_shape=jax.ShapeDtypeStruct((_B, 8, 128), jnp.float32),
        compiler_params=pltpu.CompilerParams(
            dimension_semantics=("parallel",)),
    scratch_shapes=[
            pltpu.VMEM((_C, _S * _DLP, _N), jnp.float32),
        ],
    )(adjs, feats, W_lp, W_m1, W_m2, W_dec, b_dec2)
    return out[:, 0, :_NCLS]
